# baseline (device time: 51257 ns/iter reference)
import jax
import jax.numpy as jnp
from jax import lax
from jax.experimental import pallas as pl
from jax.experimental.pallas import tpu as pltpu

N_DEV = 4
HALVES = 2


def kernel(x, w_mat, scale_x, scale_w):
    m_per, k = x.shape
    _, n = w_mat.shape
    n_per = n // N_DEV
    n_ch = n_per // HALVES
    n_remote = (N_DEV - 1) * HALVES
    n_chunks = N_DEV * HALVES

    scale = (scale_x * scale_w).reshape(1, 1)

    def body(x_ref, w_ref, s_ref, out_ref,
             wf_ref, qsend_ref, qrecv_ref, sc_send_ref, sc_recv_ref,
             w_dma_sems, send_sems, sc_send_sems, recv_sems, sc_recv_sems):
        my = lax.axis_index("i")
        s = s_ref[0, 0]

        def w_dma(c):
            blk = lax.rem(my + 1 + c // HALVES, N_DEV)
            return pltpu.make_async_copy(
                w_ref.at[:, pl.ds(blk * n_per + (c % HALVES) * n_ch, n_ch)],
                wf_ref.at[c % 2],
                w_dma_sems.at[c % 2],
            )

        w_dma(0).start()

        barrier = pltpu.get_barrier_semaphore()
        for off in range(1, N_DEV):
            pl.semaphore_signal(
                barrier, inc=1,
                device_id=(lax.rem(my + off, N_DEV),),
                device_id_type=pl.DeviceIdType.MESH,
            )
        pl.semaphore_wait(barrier, N_DEV - 1)

        for c in range(n_remote):
            jj = lax.rem(my + 1 + c // HALVES, N_DEV)
            w_dma(c).wait()
            w_dma(c + 1).start()

            acc = jnp.dot(x_ref[...], wf_ref[c % 2],
                          preferred_element_type=jnp.float32)

            a = jnp.maximum(jnp.max(jnp.abs(acc), axis=0, keepdims=True),
                            1e-20)
            qsend_ref[c] = jnp.round(acc * (127.0 / a)).astype(jnp.int8)
            sc_send_ref[c] = a * (s / 127.0)

            pltpu.make_async_remote_copy(
                src_ref=qsend_ref.at[c],
                dst_ref=qrecv_ref.at[c],
                send_sem=send_sems.at[c],
                recv_sem=recv_sems.at[c],
                device_id=(jj,),
                device_id_type=pl.DeviceIdType.MESH,
            ).start()
            pltpu.make_async_remote_copy(
                src_ref=sc_send_ref.at[c],
                dst_ref=sc_recv_ref.at[c],
                send_sem=sc_send_sems.at[c],
                recv_sem=sc_recv_sems.at[c],
                device_id=(jj,),
                device_id_type=pl.DeviceIdType.MESH,
            ).start()

        for c in range(n_remote, n_chunks):
            w_dma(c).wait()
            if c + 1 < n_chunks:
                w_dma(c + 1).start()
            acc = jnp.dot(x_ref[...], wf_ref[c % 2],
                          preferred_element_type=jnp.float32)
            out_ref[pl.ds(my * m_per, m_per),
                    (c % HALVES) * n_ch:(c % HALVES + 1) * n_ch] = (
                jnp.maximum(acc * s, 0.0))

        for c in range(n_remote):
            pltpu.make_async_remote_copy(
                src_ref=qsend_ref.at[c], dst_ref=qrecv_ref.at[c],
                send_sem=send_sems.at[c], recv_sem=recv_sems.at[c],
                device_id=(my,), device_id_type=pl.DeviceIdType.MESH,
            ).wait_recv()
            pltpu.make_async_remote_copy(
                src_ref=sc_send_ref.at[c], dst_ref=sc_recv_ref.at[c],
                send_sem=sc_send_sems.at[c], recv_sem=sc_recv_sems.at[c],
                device_id=(my,), device_id_type=pl.DeviceIdType.MESH,
            ).wait_recv()

            src = lax.rem(my + N_DEV - 1 - c // HALVES, N_DEV)
            y = qrecv_ref[c].astype(jnp.float32) * sc_recv_ref[c]
            out_ref[pl.ds(src * m_per, m_per),
                    (c % HALVES) * n_ch:(c % HALVES + 1) * n_ch] = (
                jnp.maximum(y, 0.0))

        for c in range(n_remote):
            pltpu.make_async_remote_copy(
                src_ref=qsend_ref.at[c], dst_ref=qrecv_ref.at[c],
                send_sem=send_sems.at[c], recv_sem=recv_sems.at[c],
                device_id=(my,), device_id_type=pl.DeviceIdType.MESH,
            ).wait_send()
            pltpu.make_async_remote_copy(
                src_ref=sc_send_ref.at[c], dst_ref=sc_recv_ref.at[c],
                send_sem=sc_send_sems.at[c], recv_sem=sc_recv_sems.at[c],
                device_id=(my,), device_id_type=pl.DeviceIdType.MESH,
            ).wait_send()

    return pl.pallas_call(
        body,
        out_shape=jax.ShapeDtypeStruct((N_DEV * m_per, n_per), jnp.float32),
        in_specs=[
            pl.BlockSpec(memory_space=pltpu.VMEM),
            pl.BlockSpec(memory_space=pltpu.MemorySpace.HBM),
            pl.BlockSpec(memory_space=pltpu.SMEM),
        ],
        out_specs=pl.BlockSpec(memory_space=pltpu.VMEM),
        scratch_shapes=[
            pltpu.VMEM((2, k, n_ch), jnp.float32),
            pltpu.VMEM((n_remote, m_per, n_ch), jnp.int8),
            pltpu.VMEM((n_remote, m_per, n_ch), jnp.int8),
            pltpu.VMEM((n_remote, 1, n_ch), jnp.float32),
            pltpu.VMEM((n_remote, 1, n_ch), jnp.float32),
            pltpu.SemaphoreType.DMA((2,)),
            pltpu.SemaphoreType.DMA((n_remote,)),
            pltpu.SemaphoreType.DMA((n_remote,)),
            pltpu.SemaphoreType.DMA((n_remote,)),
            pltpu.SemaphoreType.DMA((n_remote,)),
        ],
        compiler_params=pltpu.CompilerParams(
            collective_id=0,
            vmem_limit_bytes=44 * 1024 * 1024,
        ),
    )(x, w_mat, scale)


# device time: 38755 ns/iter; 1.3226x vs baseline; 1.3226x over previous
import jax
import jax.numpy as jnp
from jax import lax
from jax.experimental import pallas as pl
from jax.experimental.pallas import tpu as pltpu

N_DEV = 4


def kernel(x, w_mat, scale_x, scale_w):
    m_per, k = x.shape
    _, n = w_mat.shape
    n_per = n // N_DEV

    scale = (scale_x * scale_w).reshape(1, 1)

    def body(x_ref, w_ref, s_ref, out_ref,
             x8_ref, w8_ref, wf_ref, qsend_ref, qrecv_ref,
             sc_send_ref, sc_recv_ref,
             w_dma_sems, send_sems, sc_send_sems, recv_sems, sc_recv_sems):
        my = lax.axis_index("i")
        s = s_ref[0, 0]

        k_half = k // 2

        def w_dma(c):
            jj = lax.rem(my + 1 + c // 2, N_DEV)
            return pltpu.make_async_copy(
                w_ref.at[pl.ds((c % 2) * k_half, k_half),
                         pl.ds(jj * n_per, n_per)],
                wf_ref.at[c % 2],
                w_dma_sems.at[c % 2],
            )

        def w_load(step):
            for h in range(2):
                c = 2 * step + h
                w_dma(c).wait()
                if c + 1 < 2 * N_DEV:
                    w_dma(c + 1).start()
                w8_ref[h * k_half:(h + 1) * k_half, :] = (
                    wf_ref[c % 2].astype(jnp.float8_e4m3fn))

        w_dma(0).start()

        x8_ref[...] = x_ref[...].astype(jnp.float8_e4m3fn)

        barrier = pltpu.get_barrier_semaphore()
        for off in range(1, N_DEV):
            pl.semaphore_signal(
                barrier, inc=1,
                device_id=(lax.rem(my + off, N_DEV),),
                device_id_type=pl.DeviceIdType.MESH,
            )
        pl.semaphore_wait(barrier, N_DEV - 1)

        for step in range(N_DEV - 1):
            jj = lax.rem(my + 1 + step, N_DEV)
            w_load(step)

            acc = jnp.dot(x8_ref[...], w8_ref[...],
                          preferred_element_type=jnp.float32)

            a = jnp.maximum(jnp.max(jnp.abs(acc), axis=0, keepdims=True),
                            1e-20)
            qsend_ref[step] = jnp.round(acc * (127.0 / a)).astype(jnp.int8)
            sc_send_ref[step] = a * (s / 127.0)

            data = pltpu.make_async_remote_copy(
                src_ref=qsend_ref.at[step],
                dst_ref=qrecv_ref.at[step],
                send_sem=send_sems.at[step],
                recv_sem=recv_sems.at[step],
                device_id=(jj,),
                device_id_type=pl.DeviceIdType.MESH,
            )
            data.start()
            sc = pltpu.make_async_remote_copy(
                src_ref=sc_send_ref.at[step],
                dst_ref=sc_recv_ref.at[step],
                send_sem=sc_send_sems.at[step],
                recv_sem=sc_recv_sems.at[step],
                device_id=(jj,),
                device_id_type=pl.DeviceIdType.MESH,
            )
            sc.start()

        w_load(N_DEV - 1)
        acc = jnp.dot(x8_ref[...], w8_ref[...],
                      preferred_element_type=jnp.float32)
        out_ref[pl.ds(my * m_per, m_per), :] = jnp.maximum(acc * s, 0.0)

        for t in range(N_DEV - 1):
            wait_d = pltpu.make_async_remote_copy(
                src_ref=qsend_ref.at[t], dst_ref=qrecv_ref.at[t],
                send_sem=send_sems.at[t], recv_sem=recv_sems.at[t],
                device_id=(my,), device_id_type=pl.DeviceIdType.MESH,
            )
            wait_d.wait_recv()
            wait_s = pltpu.make_async_remote_copy(
                src_ref=sc_send_ref.at[t], dst_ref=sc_recv_ref.at[t],
                send_sem=sc_send_sems.at[t], recv_sem=sc_recv_sems.at[t],
                device_id=(my,), device_id_type=pl.DeviceIdType.MESH,
            )
            wait_s.wait_recv()

            src = lax.rem(my + 3 - t, N_DEV)
            y = qrecv_ref[t].astype(jnp.float32) * sc_recv_ref[t]
            out_ref[pl.ds(src * m_per, m_per), :] = jnp.maximum(y, 0.0)

        for t in range(N_DEV - 1):
            data = pltpu.make_async_remote_copy(
                src_ref=qsend_ref.at[t], dst_ref=qrecv_ref.at[t],
                send_sem=send_sems.at[t], recv_sem=recv_sems.at[t],
                device_id=(my,), device_id_type=pl.DeviceIdType.MESH,
            )
            data.wait_send()
            sc = pltpu.make_async_remote_copy(
                src_ref=sc_send_ref.at[t], dst_ref=sc_recv_ref.at[t],
                send_sem=sc_send_sems.at[t], recv_sem=sc_recv_sems.at[t],
                device_id=(my,), device_id_type=pl.DeviceIdType.MESH,
            )
            sc.wait_send()

    return pl.pallas_call(
        body,
        out_shape=jax.ShapeDtypeStruct((N_DEV * m_per, n_per), jnp.float32),
        in_specs=[
            pl.BlockSpec(memory_space=pltpu.VMEM),
            pl.BlockSpec(memory_space=pltpu.MemorySpace.HBM),
            pl.BlockSpec(memory_space=pltpu.SMEM),
        ],
        out_specs=pl.BlockSpec(memory_space=pltpu.VMEM),
        scratch_shapes=[
            pltpu.VMEM((m_per, k), jnp.float8_e4m3fn),
            pltpu.VMEM((k, n_per), jnp.float8_e4m3fn),
            pltpu.VMEM((2, k // 2, n_per), jnp.float32),
            pltpu.VMEM((N_DEV - 1, m_per, n_per), jnp.int8),
            pltpu.VMEM((N_DEV - 1, m_per, n_per), jnp.int8),
            pltpu.VMEM((N_DEV - 1, 1, n_per), jnp.float32),
            pltpu.VMEM((N_DEV - 1, 1, n_per), jnp.float32),
            pltpu.SemaphoreType.DMA((2,)),
            pltpu.SemaphoreType.DMA((N_DEV - 1,)),
            pltpu.SemaphoreType.DMA((N_DEV - 1,)),
            pltpu.SemaphoreType.DMA((N_DEV - 1,)),
            pltpu.SemaphoreType.DMA((N_DEV - 1,)),
        ],
        compiler_params=pltpu.CompilerParams(
            collective_id=0,
            vmem_limit_bytes=40 * 1024 * 1024,
        ),
    )(x, w_mat, scale)
